# Initial kernel scaffold; baseline (speedup 1.0000x reference)
import jax
import jax.numpy as jnp
from jax.experimental import pallas as pl


def kernel(context, attn, src_input, decoder_input, decoder_hidden, W_lin, b_lin, W_gen, b_gen):
    raise NotImplementedError("write your pallas kernel here")


# speedup vs baseline: 29.0000x; 29.0000x over previous
import functools

import jax
import jax.numpy as jnp
from jax import lax
from jax.experimental import pallas as pl
from jax.experimental.pallas import tpu as pltpu
from jax.experimental.pallas import tpu_sc as plsc

B = 1024
H = 128
V = 100000
VD = 64
SRC = 200
SP = 208

VB = 6400
NV = 16
VPAD = NV * VB
NB = 2
BB = B // NB

NW = 32
RW = B // NW
IW = RW * SP
JW = IW // 128
CH = SP // 16


def _a_body(ctx_ref, hid_ref, dec_ref, wg_ref, bg_ref, w_ref, bl_ref, att_ref,
            alpha_ref, bml_ref, amax_ref, copy_ref, m_s, z_s, ai_s, lg_s):
    j = pl.program_id(0)

    @pl.when(j == 0)
    def _init():
        ctx = ctx_ref[...]
        wg = wg_ref[...]
        z = (jnp.sum(ctx * wg[0, :H][None, :], axis=1)
             + jnp.sum(hid_ref[...] * wg[0, H:2 * H][None, :], axis=1)
             + jnp.sum(dec_ref[...] * wg[0, 2 * H:][None, :], axis=1)
             + bg_ref[...])
        g = jax.nn.sigmoid(z)
        lg_s[...] = jax.nn.log_sigmoid(z)
        copy_ref[...] = att_ref[...] * (1.0 - g)[:, None]
        m_s[...] = jnp.full((B,), -1e30, jnp.float32)
        z_s[...] = jnp.zeros((B,), jnp.float32)
        ai_s[...] = jnp.zeros((B,), jnp.int32)

    logits = lax.dot_general(ctx_ref[...], w_ref[...], (((1,), (1,)), ((), ())),
                             preferred_element_type=jnp.float32)
    logits = logits + bl_ref[0, 0, :][None, :]
    m_t = jnp.max(logits, axis=1)
    a_t = jnp.argmax(logits, axis=1).astype(jnp.int32) + j * VB
    z_s[...] = z_s[...] + jnp.sum(jnp.exp(logits), axis=1)
    upd = m_t > m_s[...]
    ai_s[...] = jnp.where(upd, a_t, ai_s[...])
    m_s[...] = jnp.maximum(m_t, m_s[...])

    @pl.when(j == NV - 1)
    def _fin():
        alpha = lg_s[...] - jnp.log(z_s[...])
        alpha_ref[...] = alpha
        bml_ref[...] = m_s[...] + alpha
        amax_ref[...] = ai_s[...]


_call_a = pl.pallas_call(
    _a_body,
    grid=(NV,),
    in_specs=[
        pl.BlockSpec((B, H), lambda j: (0, 0)),
        pl.BlockSpec((B, H), lambda j: (0, 0)),
        pl.BlockSpec((B, VD), lambda j: (0, 0)),
        pl.BlockSpec((1, 2 * H + VD), lambda j: (0, 0)),
        pl.BlockSpec((1,), lambda j: (0,)),
        pl.BlockSpec((VB, H), lambda j: (j, 0)),
        pl.BlockSpec((1, 1, VB), lambda j: (j, 0, 0)),
        pl.BlockSpec((B, SP), lambda j: (0, 0)),
    ],
    out_specs=[
        pl.BlockSpec((B,), lambda j: (0,)),
        pl.BlockSpec((B,), lambda j: (0,)),
        pl.BlockSpec((B,), lambda j: (0,)),
        pl.BlockSpec((B, SP), lambda j: (0, 0)),
    ],
    out_shape=[
        jax.ShapeDtypeStruct((B,), jnp.float32),
        jax.ShapeDtypeStruct((B,), jnp.float32),
        jax.ShapeDtypeStruct((B,), jnp.int32),
        jax.ShapeDtypeStruct((B, SP), jnp.float32),
    ],
    scratch_shapes=[
        pltpu.VMEM((B,), jnp.float32),
        pltpu.VMEM((B,), jnp.float32),
        pltpu.VMEM((B,), jnp.int32),
        pltpu.VMEM((B,), jnp.float32),
    ],
)


def _c_body(ctx_ref, w_ref, bl_ref, alpha_ref, out_ref):
    logits = lax.dot_general(ctx_ref[...], w_ref[...], (((1,), (1,)), ((), ())),
                             preferred_element_type=jnp.float32)
    out_ref[...] = logits + bl_ref[0, 0, :][None, :] + alpha_ref[...][:, None]


_call_c = pl.pallas_call(
    _c_body,
    grid=(NV, NB),
    in_specs=[
        pl.BlockSpec((BB, H), lambda j, b: (b, 0)),
        pl.BlockSpec((VB, H), lambda j, b: (j, 0)),
        pl.BlockSpec((1, 1, VB), lambda j, b: (j, 0, 0)),
        pl.BlockSpec((BB,), lambda j, b: (b,)),
    ],
    out_specs=pl.BlockSpec((BB, VB), lambda j, b: (b, j)),
    out_shape=jax.ShapeDtypeStruct((B, V), jnp.float32),
)


_sc_mesh = plsc.VectorSubcoreMesh(core_axis_name="c", subcore_axis_name="s")


@functools.partial(
    pl.kernel,
    mesh=_sc_mesh,
    compiler_params=pltpu.CompilerParams(needs_layout_passes=False),
    out_type=[
        jax.ShapeDtypeStruct((NW, JW, 128), jnp.float32),
        jax.ShapeDtypeStruct((NW, IW), jnp.float32),
    ],
    scratch_types=[
        pltpu.VMEM((JW, 128), jnp.int32),
        pltpu.VMEM((IW,), jnp.float32),
        pltpu.VMEM((IW,), jnp.float32),
        pltpu.VMEM((JW, 128), jnp.float32),
        pltpu.VMEM((V,), jnp.float32),
        pltpu.SemaphoreType.DMA,
    ],
)
def _sc_gather(outflat, src3, copy2, base3, ct2,
               flat_v, copy_v, ct_v, base_v, dense, sem):
    wid = lax.axis_index("s") * 2 + lax.axis_index("c")
    pltpu.sync_copy(src3.at[wid], flat_v)
    pltpu.sync_copy(copy2.at[wid], copy_v)

    def build_row(r, carry):
        base_flat = (wid * RW + r) * V
        for c in range(CH):
            off = r * SP + c * 16
            ri, ci = off // 128, off % 128
            flat_v[ri, pl.ds(ci, 16)] = flat_v[ri, pl.ds(ci, 16)] + base_flat
        return carry

    lax.fori_loop(0, RW, build_row, 0)

    copies = [pltpu.async_copy(outflat.at[flat_v.at[j2]], base_v.at[j2], sem)
              for j2 in range(JW)]

    def dedup_row(r, carry):
        base_flat = (wid * RW + r) * V
        for c in range(CH):
            off = r * SP + c * 16
            vocab16 = flat_v[off // 128, pl.ds(off % 128, 16)] - base_flat
            plsc.store_scatter(dense, [vocab16], jnp.zeros((16,), jnp.float32))
        for c in range(CH):
            off = r * SP + c * 16
            vocab16 = flat_v[off // 128, pl.ds(off % 128, 16)] - base_flat
            plsc.addupdate_scatter(dense, [vocab16], copy_v[pl.ds(off, 16)])
        for c in range(CH):
            off = r * SP + c * 16
            vocab16 = flat_v[off // 128, pl.ds(off % 128, 16)] - base_flat
            ct_v[pl.ds(off, 16)] = plsc.load_gather(dense, [vocab16])
        return carry

    lax.fori_loop(0, RW, dedup_row, 0)

    for cpy in copies:
        cpy.wait()
    pltpu.sync_copy(base_v, base3.at[wid])
    pltpu.sync_copy(ct_v, ct2.at[wid])


def _d_body(base_ref, ct_ref, srcp_ref, bml_ref, amax_ref, corr_ref, sym_ref):
    corrected = jnp.log(jnp.exp(base_ref[...]) + ct_ref[...])
    corr_ref[...] = corrected
    cand_v = jnp.max(corrected, axis=1)
    cand_s = jnp.argmax(corrected, axis=1).astype(jnp.int32)
    iot = lax.broadcasted_iota(jnp.int32, (B, SP), 1)
    cidx = jnp.sum(jnp.where(iot == cand_s[:, None], srcp_ref[...], 0), axis=1)
    am = amax_ref[...]
    bml = bml_ref[...]
    sym_ref[...] = jnp.where(
        cand_v > bml, cidx,
        jnp.where(cand_v == bml, jnp.minimum(cidx, am), am))


_call_d = pl.pallas_call(
    _d_body,
    out_shape=[
        jax.ShapeDtypeStruct((B, SP), jnp.float32),
        jax.ShapeDtypeStruct((B,), jnp.int32),
    ],
)


@functools.partial(
    pl.kernel,
    mesh=_sc_mesh,
    compiler_params=pltpu.CompilerParams(needs_layout_passes=False),
    out_type=[],
    scratch_types=[
        pltpu.VMEM((JW, 128), jnp.int32),
        pltpu.VMEM((JW, 128), jnp.float32),
        pltpu.SemaphoreType.DMA,
    ],
)
def _sc_scatter(outflat, src3, corr3, flat_v, corr_v, sem):
    wid = lax.axis_index("s") * 2 + lax.axis_index("c")
    pltpu.sync_copy(src3.at[wid], flat_v)
    pltpu.sync_copy(corr3.at[wid], corr_v)

    def build_row(r, carry):
        base_flat = (wid * RW + r) * V
        for c in range(CH):
            off = r * SP + c * 16
            ri, ci = off // 128, off % 128
            flat_v[ri, pl.ds(ci, 16)] = flat_v[ri, pl.ds(ci, 16)] + base_flat
        return carry

    lax.fori_loop(0, RW, build_row, 0)

    copies = [pltpu.async_copy(corr_v.at[j2], outflat.at[flat_v.at[j2]], sem)
              for j2 in range(JW)]
    for cpy in copies:
        cpy.wait()


def kernel(context, attn, src_input, decoder_input, decoder_hidden,
           W_lin, b_lin, W_gen, b_gen):
    ctx = context.reshape(B, H)
    hid = decoder_hidden[0, 0]
    dec = decoder_input.reshape(B, VD)
    att = attn.reshape(B, SRC)
    src = src_input.astype(jnp.int32)
    srcp = jnp.concatenate(
        [src, jnp.broadcast_to(src[:, SRC - 1:SRC], (B, SP - SRC))], axis=1)
    attp = jnp.concatenate([att, jnp.zeros((B, SP - SRC), att.dtype)], axis=1)

    wpad = jnp.zeros((VPAD, H), W_lin.dtype).at[:V].set(W_lin)
    bl3 = jnp.full((VPAD,), -1e30, b_lin.dtype).at[:V].set(b_lin).reshape(NV, 1, VB)
    alpha, bml, amax, copy = _call_a(ctx, hid, dec, W_gen, b_gen,
                                     wpad, bl3, attp)
    out2d = _call_c(ctx, wpad, bl3, alpha)

    r = jax.new_ref(out2d.reshape(B * V))
    src3 = srcp.reshape(NW, JW, 128)
    base3, ct2 = _sc_gather(r, src3, copy.reshape(NW, IW))
    corr, sym = _call_d(base3.reshape(B, SP), ct2.reshape(B, SP),
                        srcp, bml, amax)
    _sc_scatter(r, src3, corr.reshape(NW, JW, 128))
    out = jax.freeze(r)
    return out.reshape(B, 1, V), sym.reshape(B, 1, 1)
